# Initial kernel scaffold; baseline (speedup 1.0000x reference)
import jax
import jax.numpy as jnp
from jax.experimental import pallas as pl


def kernel(q, k, v):
    raise NotImplementedError("write your pallas kernel here")


# speedup vs baseline: 82.1663x; 82.1663x over previous
import functools
import math

import jax
import jax.numpy as jnp
from jax.experimental import pallas as pl
from jax.experimental.pallas import tpu as pltpu

_TOPK_FRAC = 0.05


def _fused_body(q_ref, k_ref, v_ref, o_ref, *, k_top, S, D):
    q = q_ref[0]
    k = k_ref[0]
    v = v_ref[0]

    q_bin = jnp.sign(q).astype(jnp.bfloat16)
    k_bin = jnp.sign(k).astype(jnp.bfloat16)
    s_int = jax.lax.dot_general(
        q_bin, k_bin, (((1,), (1,)), ((), ())),
        preferred_element_type=jnp.float32)

    col = jax.lax.broadcasted_iota(jnp.int32, s_int.shape, 1).astype(
        jnp.float32)
    ckey = s_int * float(S) + (float(S - 1) - col)

    lo0 = -float(D * S + 1)
    hi0 = float(D * S + S)
    span = int(hi0 - lo0)
    iters = max(1, math.ceil(math.log2(span)))
    bq = q.shape[0]
    lo_init = jnp.full((bq, 1), lo0, jnp.float32)
    hi_init = jnp.full((bq, 1), hi0, jnp.float32)

    def body(_, carry):
        lo, hi = carry
        mid = jnp.floor((lo + hi) * 0.5)
        cnt = jnp.sum((ckey >= mid).astype(jnp.float32), axis=1,
                      keepdims=True)
        ge = cnt >= float(k_top)
        return jnp.where(ge, mid, lo), jnp.where(ge, hi, mid)

    lo, _ = jax.lax.fori_loop(0, iters, body, (lo_init, hi_init))
    mask = ckey >= lo

    ps = jax.lax.dot_general(
        q, k, (((1,), (1,)), ((), ())),
        preferred_element_type=jnp.float32,
        precision=jax.lax.Precision.HIGHEST) * (1.0 / math.sqrt(D))
    psm = jnp.where(mask, ps, -jnp.inf)
    m = jnp.max(psm, axis=1, keepdims=True)
    e = jnp.exp(psm - m)
    denom = jnp.sum(e, axis=1, keepdims=True)
    out = jax.lax.dot_general(
        e, v, (((1,), (0,)), ((), ())),
        preferred_element_type=jnp.float32,
        precision=jax.lax.Precision.HIGHEST) / denom
    o_ref[0] = out


def kernel(q, k, v):
    B, H, S, D = q.shape
    k_top = max(1, int(S * _TOPK_FRAC))
    BQ = 256
    qr = q.reshape(B * H, S, D)
    kr = k.reshape(B * H, S, D)
    vr = v.reshape(B * H, S, D)
    grid = (B * H, S // BQ)
    out = pl.pallas_call(
        functools.partial(_fused_body, k_top=k_top, S=S, D=D),
        grid=grid,
        in_specs=[
            pl.BlockSpec((1, BQ, D), lambda h, i: (h, i, 0)),
            pl.BlockSpec((1, S, D), lambda h, i: (h, 0, 0)),
            pl.BlockSpec((1, S, D), lambda h, i: (h, 0, 0)),
        ],
        out_specs=pl.BlockSpec((1, BQ, D), lambda h, i: (h, i, 0)),
        out_shape=jax.ShapeDtypeStruct((B * H, S, D), jnp.float32),
        compiler_params=pltpu.CompilerParams(
            dimension_semantics=("arbitrary", "arbitrary")),
    )(qr, kr, vr)
    return out.reshape(B, H, S, D)


# speedup vs baseline: 122.2006x; 1.4872x over previous
import functools
import math

import jax
import jax.numpy as jnp
from jax.experimental import pallas as pl
from jax.experimental.pallas import tpu as pltpu

_TOPK_FRAC = 0.05


def _dot3(a, b, contract):
    dims = (contract, ((), ()))
    a_hi = a.astype(jnp.bfloat16)
    a_lo = (a - a_hi.astype(jnp.float32)).astype(jnp.bfloat16)
    b_hi = b.astype(jnp.bfloat16)
    b_lo = (b - b_hi.astype(jnp.float32)).astype(jnp.bfloat16)
    f32 = jnp.float32
    return (jax.lax.dot_general(a_hi, b_hi, dims, preferred_element_type=f32)
            + jax.lax.dot_general(a_hi, b_lo, dims, preferred_element_type=f32)
            + jax.lax.dot_general(a_lo, b_hi, dims, preferred_element_type=f32))


def _fused_body(q_ref, k_ref, v_ref, o_ref, *, k_top, S, D):
    q = q_ref[0]
    k = k_ref[0]
    v = v_ref[0]

    q_bin = jnp.sign(q).astype(jnp.bfloat16)
    k_bin = jnp.sign(k).astype(jnp.bfloat16)
    s_int = jax.lax.dot_general(
        q_bin, k_bin, (((1,), (1,)), ((), ())),
        preferred_element_type=jnp.float32)

    col = jax.lax.broadcasted_iota(jnp.int32, s_int.shape, 1).astype(
        jnp.float32)
    ckey = s_int * float(S) + (float(S - 1) - col)

    lo0 = -float(D * S + 1)
    hi0 = float(D * S + S)
    span = int(hi0 - lo0)
    iters = max(1, math.ceil(math.log2(span)))
    bq = q.shape[0]
    lo_init = jnp.full((bq, 1), lo0, jnp.float32)
    hi_init = jnp.full((bq, 1), hi0, jnp.float32)

    def body(_, carry):
        lo, hi = carry
        mid = jnp.floor((lo + hi) * 0.5)
        cnt = jnp.sum((ckey >= mid).astype(jnp.float32), axis=1,
                      keepdims=True)
        ge = cnt >= float(k_top)
        return jnp.where(ge, mid, lo), jnp.where(ge, hi, mid)

    lo, _ = jax.lax.fori_loop(0, iters, body, (lo_init, hi_init))
    mask = ckey >= lo

    ps = _dot3(q, k, ((1,), (1,))) * (1.0 / math.sqrt(D))
    psm = jnp.where(mask, ps, -jnp.inf)
    m = jnp.max(psm, axis=1, keepdims=True)
    e = jnp.exp(psm - m)
    denom = jnp.sum(e, axis=1, keepdims=True)
    out = _dot3(e, v, ((1,), (0,))) / denom
    o_ref[0] = out


def kernel(q, k, v):
    B, H, S, D = q.shape
    k_top = max(1, int(S * _TOPK_FRAC))
    BQ = 256
    qr = q.reshape(B * H, S, D)
    kr = k.reshape(B * H, S, D)
    vr = v.reshape(B * H, S, D)
    grid = (B * H, S // BQ)
    out = pl.pallas_call(
        functools.partial(_fused_body, k_top=k_top, S=S, D=D),
        grid=grid,
        in_specs=[
            pl.BlockSpec((1, BQ, D), lambda h, i: (h, i, 0)),
            pl.BlockSpec((1, S, D), lambda h, i: (h, 0, 0)),
            pl.BlockSpec((1, S, D), lambda h, i: (h, 0, 0)),
        ],
        out_specs=pl.BlockSpec((1, BQ, D), lambda h, i: (h, i, 0)),
        out_shape=jax.ShapeDtypeStruct((B * H, S, D), jnp.float32),
        compiler_params=pltpu.CompilerParams(
            dimension_semantics=("arbitrary", "arbitrary")),
    )(qr, kr, vr)
    return out.reshape(B, H, S, D)
